# baseline (speedup 1.0000x reference)
import functools

import jax
import jax.numpy as jnp
from jax import lax
from jax.experimental import pallas as pl
from jax.experimental.pallas import tpu as pltpu
from jax.experimental.pallas import tpu_sc as plsc

_NUM_DAYS = 366
_HIDDEN = 128
_BATCH = 4096
_HIST = 200
_N = _BATCH * _HIST
_NC = 2
_NS = 16
_NW = _NC * _NS
_BPW = _N // _NW
_CH = 128
_GPS = 2
_SCH = _CH * _GPS
_NSCH = _BPW // _SCH
_NB = 3
_D = 1


def _make_gather():
    mesh = plsc.VectorSubcoreMesh(core_axis_name="c", subcore_axis_name="s")

    @functools.partial(
        pl.kernel,
        mesh=mesh,
        out_type=jax.ShapeDtypeStruct((_N, _HIDDEN), jnp.float32),
        scratch_types=[
            pltpu.VMEM((_NSCH * _GPS, _CH), jnp.int32),
            pltpu.VMEM((_NB, _SCH, _HIDDEN), jnp.float32),
            pltpu.VMEM_SHARED((_NUM_DAYS, _HIDDEN), jnp.float32),
        ]
        + [pltpu.SemaphoreType.DMA] * (2 * _NB),
    )
    def k(table_hbm, idx_hbm, out_hbm, idx_v, rows_v, table_sp, *sems):
        gsems = sems[:_NB]
        wsems = sems[_NB:]
        sid = lax.axis_index("s")
        wid = sid * _NC + lax.axis_index("c")
        base = wid * _BPW

        @pl.when(sid == 0)
        def _stage_table():
            pltpu.sync_copy(table_hbm, table_sp)

        nrow = _NSCH * _GPS
        pltpu.sync_copy(idx_hbm.at[pl.ds(wid * nrow, nrow)], idx_v)
        plsc.subcore_barrier()

        def gather_start(j, b):
            for g in range(_GPS):
                pltpu.async_copy(
                    table_sp.at[idx_v.at[j * _GPS + g]],
                    rows_v.at[b].at[pl.ds(g * _CH, _CH)],
                    gsems[b],
                )

        def gather_wait(j, b):
            for g in range(_GPS):
                pltpu.make_async_copy(
                    table_sp.at[idx_v.at[j * _GPS + g]],
                    rows_v.at[b].at[pl.ds(g * _CH, _CH)],
                    gsems[b],
                ).wait()

        def write_start(j, b):
            pltpu.async_copy(
                rows_v.at[b], out_hbm.at[pl.ds(base + j * _SCH, _SCH)], wsems[b]
            )

        def write_wait(b):
            pltpu.make_async_copy(
                rows_v.at[b], out_hbm.at[pl.ds(base, _SCH)], wsems[b]
            ).wait()

        for b in range(_D):
            gather_start(b, b)

        def group(grp, carry):
            j0 = grp * _NB
            for b in range(_NB):
                jp = j0 + b + _D
                bp = (b + _D) % _NB

                @pl.when(jp < _NSCH)
                def _prefetch(jp=jp, bp=bp):
                    @pl.when(jp - _NB >= 0)
                    def _drain():
                        write_wait(bp)

                    gather_start(jp, bp)

                j = j0 + b
                gather_wait(j, b)
                write_start(j, b)
            return carry

        ngroup = _NSCH // _NB
        lax.fori_loop(0, ngroup, group, 0)

        assert _NSCH - ngroup * _NB <= _D
        for r in range(ngroup * _NB, _NSCH):
            b = r % _NB
            gather_wait(r, b)
            write_start(r, b)

        for b in range(_NB):
            write_wait(b)

    return k


_gather = _make_gather()


def kernel(days, emb_weight):
    idx = days.reshape(_N // _CH, _CH)
    out = _gather(emb_weight, idx)
    return out.reshape(_BATCH, _HIST, _HIDDEN)


# speedup vs baseline: 1.0648x; 1.0648x over previous
import functools

import jax
import jax.numpy as jnp
from jax import lax
from jax.experimental import pallas as pl
from jax.experimental.pallas import tpu as pltpu
from jax.experimental.pallas import tpu_sc as plsc

_NUM_DAYS = 366
_HIDDEN = 128
_BATCH = 4096
_HIST = 200
_N = _BATCH * _HIST
_NC = 2
_NS = 16
_NW = _NC * _NS
_BPW = _N // _NW
_CH = 128
_NCHUNK = _BPW // _CH
_NB = 6
_D = 3


def _make_gather():
    mesh = plsc.VectorSubcoreMesh(core_axis_name="c", subcore_axis_name="s")

    @functools.partial(
        pl.kernel,
        mesh=mesh,
        out_type=jax.ShapeDtypeStruct((_N, _HIDDEN), jnp.float32),
        scratch_types=[
            pltpu.VMEM((_NCHUNK, _CH), jnp.int32),
            pltpu.VMEM((_NB, _CH, _HIDDEN), jnp.float32),
            pltpu.VMEM_SHARED((_NUM_DAYS, _HIDDEN), jnp.float32),
        ]
        + [pltpu.SemaphoreType.DMA] * (2 * _NB),
    )
    def k(table_hbm, idx_hbm, out_hbm, idx_v, rows_v, table_sp, *sems):
        gsems = sems[:_NB]
        wsems = sems[_NB:]
        sid = lax.axis_index("s")
        wid = sid * _NC + lax.axis_index("c")
        base = wid * _BPW

        @pl.when(sid == 0)
        def _stage_table():
            pltpu.sync_copy(table_hbm, table_sp)

        pltpu.sync_copy(idx_hbm.at[pl.ds(wid * _NCHUNK, _NCHUNK)], idx_v)
        plsc.subcore_barrier()

        def gather_start(j, b):
            pltpu.async_copy(table_sp.at[idx_v.at[j]], rows_v.at[b], gsems[b])

        def gather_wait(j, b):
            pltpu.make_async_copy(
                table_sp.at[idx_v.at[j]], rows_v.at[b], gsems[b]
            ).wait()

        def write_start(j, b):
            pltpu.async_copy(
                rows_v.at[b], out_hbm.at[pl.ds(base + j * _CH, _CH)], wsems[b]
            )

        def write_wait(b):
            pltpu.make_async_copy(
                rows_v.at[b], out_hbm.at[pl.ds(base, _CH)], wsems[b]
            ).wait()

        for b in range(_D):
            gather_start(b, b)

        def group(g, carry):
            j0 = g * _NB
            for b in range(_NB):
                jp = j0 + b + _D
                bp = (b + _D) % _NB

                @pl.when(jp < _NCHUNK)
                def _prefetch(jp=jp, bp=bp):
                    @pl.when(jp - _NB >= 0)
                    def _drain():
                        write_wait(bp)

                    gather_start(jp, bp)

                j = j0 + b
                gather_wait(j, b)
                write_start(j, b)
            return carry

        ngroup = _NCHUNK // _NB
        lax.fori_loop(0, ngroup, group, 0)

        assert _NCHUNK - ngroup * _NB <= _D
        for r in range(ngroup * _NB, _NCHUNK):
            b = r % _NB
            gather_wait(r, b)
            write_start(r, b)

        for b in range(_NB):
            write_wait(b)

    return k


_gather = _make_gather()


def kernel(days, emb_weight):
    idx = days.reshape(_N // _CH, _CH)
    out = _gather(emb_weight, idx)
    return out.reshape(_BATCH, _HIST, _HIDDEN)


# speedup vs baseline: 1.0691x; 1.0040x over previous
import functools

import jax
import jax.numpy as jnp
from jax import lax
from jax.experimental import pallas as pl
from jax.experimental.pallas import tpu as pltpu
from jax.experimental.pallas import tpu_sc as plsc

_NUM_DAYS = 366
_HIDDEN = 128
_BATCH = 4096
_HIST = 200
_N = _BATCH * _HIST
_NC = 2
_NS = 16
_NW = _NC * _NS
_BPW = _N // _NW
_CH = 128
_NCHUNK = _BPW // _CH
_NB = 6
_D = 3


def _make_gather():
    mesh = plsc.VectorSubcoreMesh(core_axis_name="c", subcore_axis_name="s")

    @functools.partial(
        pl.kernel,
        mesh=mesh,
        out_type=jax.ShapeDtypeStruct((_N, _HIDDEN), jnp.float32),
        scratch_types=[
            pltpu.VMEM((_NCHUNK, _CH), jnp.int32),
            pltpu.VMEM((_NB, _CH, _HIDDEN), jnp.float32),
            pltpu.VMEM_SHARED((_NUM_DAYS, _HIDDEN), jnp.float32),
        ]
        + [pltpu.SemaphoreType.DMA] * (2 * _NB),
    )
    def k(table_hbm, idx_hbm, out_hbm, idx_v, rows_v, table_sp, *sems):
        gsems = sems[:_NB]
        wsems = sems[_NB:]
        sid = lax.axis_index("s")
        wid = sid * _NC + lax.axis_index("c")
        base = wid * _BPW

        @pl.when(sid == 0)
        def _stage_table():
            pltpu.sync_copy(table_hbm, table_sp)

        pltpu.sync_copy(idx_hbm.at[pl.ds(wid * _NCHUNK, _NCHUNK)], idx_v)
        plsc.subcore_barrier()

        def gather_start(j, b):
            pltpu.async_copy(table_sp.at[idx_v.at[j]], rows_v.at[b], gsems[b])

        def gather_wait(j, b):
            pltpu.make_async_copy(
                table_sp.at[idx_v.at[j]], rows_v.at[b], gsems[b]
            ).wait()

        def write_start(j, b):
            pltpu.async_copy(
                rows_v.at[b], out_hbm.at[pl.ds(base + j * _CH, _CH)], wsems[b]
            )

        def write_wait(b):
            pltpu.make_async_copy(
                rows_v.at[b], out_hbm.at[pl.ds(base, _CH)], wsems[b]
            ).wait()

        for b in range(_D):
            gather_start(b, b)

        def group(g, carry):
            j0 = g * _NB
            for b in range(_NB):
                j = j0 + b
                gather_wait(j, b)
                write_start(j, b)

                jp = j0 + b + _D
                bp = (b + _D) % _NB

                @pl.when(jp < _NCHUNK)
                def _prefetch(jp=jp, bp=bp):
                    @pl.when(jp - _NB >= 0)
                    def _drain():
                        write_wait(bp)

                    gather_start(jp, bp)
            return carry

        ngroup = _NCHUNK // _NB
        lax.fori_loop(0, ngroup, group, 0)

        assert _NCHUNK - ngroup * _NB <= _D
        for r in range(ngroup * _NB, _NCHUNK):
            b = r % _NB
            gather_wait(r, b)
            write_start(r, b)

        for b in range(_NB):
            write_wait(b)

    return k


_gather = _make_gather()


def kernel(days, emb_weight):
    idx = days.reshape(_N // _CH, _CH)
    out = _gather(emb_weight, idx)
    return out.reshape(_BATCH, _HIST, _HIDDEN)
